# baseline (speedup 1.0000x reference)
import functools

import numpy as np
import jax
import jax.numpy as jnp
from jax import lax
from jax.experimental import pallas as pl
from jax.experimental.pallas import tpu as pltpu
from jax.experimental.pallas import tpu_sc as plsc

H_FFT = 384
W_FFT = 193
NUM_BANDS = 6
HIDDEN = 128
F = H_FFT * W_FFT
ROWS = 4 * 96
NC, NS = 2, 16
NW = NC * NS
ROWS_PER_W = ROWS // NW
PIECES = 4
PW = F // PIECES
PAIRS = PW // 32
NBANKS = 8
ACCW = NBANKS * 96
TCB = 4096


def _band_tables():
    yy = np.arange(H_FFT, dtype=np.float32).reshape(-1, 1)
    xx = np.arange(W_FFT, dtype=np.float32).reshape(1, -1)
    ry = yy / max(H_FFT - 1, 1)
    rx = xx / max(W_FFT - 1, 1)
    r = np.sqrt(ry ** 2 + rx ** 2)
    r = r / (r.max() + 1e-8)
    band = np.minimum(np.floor(r * NUM_BANDS), NUM_BANDS - 1)
    band = band.astype(np.int32).reshape(-1)
    counts = np.zeros(NUM_BANDS, dtype=np.float32)
    for b in range(NUM_BANDS):
        counts[b] = max(float((band == b).sum()), 1.0)
    inv = np.float32(1.0) / (counts + np.float32(1e-6))
    sidx = band * 16 + (np.arange(F, dtype=np.int32) % 16)
    s = sidx.reshape(-1, 2, 16)
    spk = (s[:, 0, :] | (s[:, 1, :] << 16)).reshape(-1).astype(np.int32)
    onehot = np.zeros((16, F), dtype=np.float32)
    onehot[band, np.arange(F)] = 1.0
    return spk, [float(v) for v in inv], onehot


_SPK_NP, _INV_COUNTS, _ONEHOT_NP = _band_tables()

_MESH = plsc.VectorSubcoreMesh(core_axis_name="c", subcore_axis_name="s")


@functools.partial(
    pl.kernel,
    out_type=jax.ShapeDtypeStruct((ROWS * 16,), jnp.float32),
    mesh=_MESH,
    compiler_params=pltpu.CompilerParams(
        use_tc_tiling_on_sc=False, needs_layout_passes=False),
    scratch_types=[
        pltpu.VMEM((F // 2,), jnp.int32),
        pltpu.VMEM((PW,), jnp.float32),
        pltpu.VMEM((PW,), jnp.float32),
        pltpu.VMEM((PW,), jnp.float32),
        pltpu.VMEM((PW,), jnp.float32),
        pltpu.VMEM((ACCW,), jnp.float32),
        pltpu.VMEM((ROWS_PER_W * 16,), jnp.float32),
        pltpu.VMEM((NUM_BANDS * HIDDEN,), jnp.float32),
        pltpu.VMEM((HIDDEN,), jnp.float32),
        pltpu.VMEM((HIDDEN * 16,), jnp.float32),
        pltpu.VMEM((16,), jnp.float32),
        pltpu.SemaphoreType.DMA,
        pltpu.SemaphoreType.DMA,
        pltpu.SemaphoreType.DMA,
        pltpu.SemaphoreType.DMA,
    ],
)
def _rbg(feat_hbm, spk_hbm, w1_hbm, b1_hbm, w2_hbm, b2_hbm, gate_hbm,
         spk_v, fb0, fb1, fb2, fb3, acc_v, gflat, w1v, b1v, w2v, b2v,
         semf0, semf1, semf2, semf3):
    wid = lax.axis_index("s") * NC + lax.axis_index("c")

    ih = [pltpu.async_copy(spk_hbm, spk_v, semf0),
          pltpu.async_copy(w1_hbm, w1v, semf0),
          pltpu.async_copy(b1_hbm, b1v, semf0),
          pltpu.async_copy(w2_hbm, w2v, semf0),
          pltpu.async_copy(b2_hbm, b2v, semf0)]
    for h in ih:
        h.wait()

    zero16 = jnp.zeros((16,), jnp.float32)
    fbufs = (fb0, fb1, fb2, fb3)
    fsems = (semf0, semf1, semf2, semf3)

    for w in range(ACCW // 16):
        acc_v[pl.ds(w * 16, 16)] = zero16

    def hsum(v):
        s = v[0]
        for l in range(1, 16):
            s = s + v[l]
        return s

    base = wid * ROWS_PER_W

    def mlp_gate(sums):
        means = [sums[k] * _INV_COUNTS[k] for k in range(NUM_BANDS)]
        h_chunks = []
        for c8 in range(HIDDEN // 16):
            hv = b1v[pl.ds(c8 * 16, 16)]
            for k in range(NUM_BANDS):
                hv = hv + means[k] * w1v[pl.ds(k * HIDDEN + c8 * 16, 16)]
            h_chunks.append(jnp.maximum(hv, 0.0))
        parts = [b2v[...], zero16, zero16, zero16]
        for c8 in range(HIDDEN // 16):
            for l in range(16):
                j = c8 * 16 + l
                parts[l % 4] = (
                    parts[l % 4] + h_chunks[c8][l] * w2v[pl.ds(j * 16, 16)])
        av = (parts[0] + parts[1]) + (parts[2] + parts[3])
        return 1.0 / (1.0 + jnp.exp(-av))

    def collect_sums():
        sums = []
        for k in range(NUM_BANDS):
            a = acc_v[pl.ds(k * 16, 16)]
            for b in range(1, NBANKS):
                a = a + acc_v[pl.ds(b * 96 + k * 16, 16)]
            sums.append(hsum(a))
        for w in range(ACCW // 16):
            acc_v[pl.ds(w * 16, 16)] = zero16
        return sums

    def row_body(r, carry):
        row = base + r
        for p in range(PIECES):
            pltpu.make_async_copy(
                feat_hbm.at[row, pl.ds(p * PW, PW)],
                fbufs[p], fsems[p]).wait()
            fb = fbufs[p]

            def pair_body(i, _p=p, _fb=fb):
                off = i * 32
                pk = spk_v[pl.ds(_p * (PW // 2) + i * 16, 16)]
                s0 = pk & 0xFFFF
                s1 = lax.shift_right_logical(pk, 16)
                fv0 = _fb[pl.ds(off, 16)]
                fv1 = _fb[pl.ds(off + 16, 16)]
                b0 = (i & 3) * 192
                plsc.addupdate_scatter(acc_v, [s0 + b0], fv0)
                plsc.addupdate_scatter(acc_v, [s1 + (b0 + 96)], fv1)

            plsc.parallel_loop(0, PAIRS, unroll=4)(pair_body)

            @pl.when(r < ROWS_PER_W - 1)
            def _issue_next(_p=p, _row=row):
                pltpu.async_copy(
                    feat_hbm.at[_row + 1, pl.ds(_p * PW, PW)],
                    fbufs[_p], fsems[_p])

        gflat[pl.ds(r * 16, 16)] = mlp_gate(collect_sums())
        return carry

    for q in range(PIECES):
        pltpu.async_copy(feat_hbm.at[base, pl.ds(q * PW, PW)],
                         fbufs[q], fsems[q])
    lax.fori_loop(0, ROWS_PER_W, row_body, 0)
    pltpu.sync_copy(gflat, gate_hbm.at[pl.ds(wid * (ROWS_PER_W * 16),
                                             ROWS_PER_W * 16)])


def _expand_body(g_ref, m_ref, o_ref):
    o_ref[...] = jnp.dot(g_ref[...], m_ref[...],
                         preferred_element_type=jnp.float32)


_expand_tc = pl.pallas_call(
    _expand_body,
    grid=(pl.cdiv(F, TCB),),
    in_specs=[
        pl.BlockSpec((ROWS, 16), lambda i: (0, 0)),
        pl.BlockSpec((16, TCB), lambda i: (0, i)),
    ],
    out_specs=pl.BlockSpec((ROWS, TCB), lambda i: (0, i)),
    out_shape=jax.ShapeDtypeStruct((ROWS, F), jnp.float32),
)


def kernel(feat_flat, W1, b1, W2, b2):
    B, C, Fdim = feat_flat.shape
    feat2 = feat_flat.reshape(B * C, Fdim)
    w2p = jnp.zeros((HIDDEN, 16), W2.dtype).at[:, :NUM_BANDS].set(W2)
    b2p = jnp.zeros((16,), b2.dtype).at[:NUM_BANDS].set(b2)
    gates = _rbg(feat2, jnp.asarray(_SPK_NP), W1.reshape(-1), b1,
                 w2p.reshape(-1), b2p)
    out = _expand_tc(gates.reshape(ROWS, 16), jnp.asarray(_ONEHOT_NP))
    return out.reshape(B, C, Fdim)


# speedup vs baseline: 1.0073x; 1.0073x over previous
import functools

import numpy as np
import jax
import jax.numpy as jnp
from jax import lax
from jax.experimental import pallas as pl
from jax.experimental.pallas import tpu as pltpu
from jax.experimental.pallas import tpu_sc as plsc

H_FFT = 384
W_FFT = 193
NUM_BANDS = 6
HIDDEN = 128
F = H_FFT * W_FFT
ROWS = 4 * 96
NC, NS = 2, 16
NW = NC * NS
ROWS_PER_W = ROWS // NW
PIECES = 4
PW = F // PIECES
PAIRS = PW // 32
NBANKS = 8
ACCW = NBANKS * 96
TCB = 4096


def _band_tables():
    yy = np.arange(H_FFT, dtype=np.float32).reshape(-1, 1)
    xx = np.arange(W_FFT, dtype=np.float32).reshape(1, -1)
    ry = yy / max(H_FFT - 1, 1)
    rx = xx / max(W_FFT - 1, 1)
    r = np.sqrt(ry ** 2 + rx ** 2)
    r = r / (r.max() + 1e-8)
    band = np.minimum(np.floor(r * NUM_BANDS), NUM_BANDS - 1)
    band = band.astype(np.int32).reshape(-1)
    counts = np.zeros(NUM_BANDS, dtype=np.float32)
    for b in range(NUM_BANDS):
        counts[b] = max(float((band == b).sum()), 1.0)
    inv = np.float32(1.0) / (counts + np.float32(1e-6))
    sidx = band * 16 + (np.arange(F, dtype=np.int32) % 16)
    s = sidx.reshape(-1, 2, 16)
    spk = (s[:, 0, :] | (s[:, 1, :] << 16)).reshape(-1).astype(np.int32)
    onehot = np.zeros((16, F), dtype=np.float32)
    onehot[band, np.arange(F)] = 1.0
    return spk, [float(v) for v in inv], onehot


_SPK_NP, _INV_COUNTS, _ONEHOT_NP = _band_tables()

_MESH = plsc.VectorSubcoreMesh(core_axis_name="c", subcore_axis_name="s")


@functools.partial(
    pl.kernel,
    out_type=jax.ShapeDtypeStruct((ROWS * 16,), jnp.float32),
    mesh=_MESH,
    compiler_params=pltpu.CompilerParams(
        use_tc_tiling_on_sc=False, needs_layout_passes=False),
    scratch_types=[
        pltpu.VMEM((F // 2,), jnp.int32),
        pltpu.VMEM((PW,), jnp.float32),
        pltpu.VMEM((PW,), jnp.float32),
        pltpu.VMEM((ACCW,), jnp.float32),
        pltpu.VMEM((ROWS_PER_W * 16,), jnp.float32),
        pltpu.VMEM((NUM_BANDS * HIDDEN,), jnp.float32),
        pltpu.VMEM((HIDDEN,), jnp.float32),
        pltpu.VMEM((HIDDEN * 16,), jnp.float32),
        pltpu.VMEM((16,), jnp.float32),
        pltpu.SemaphoreType.DMA,
        pltpu.SemaphoreType.DMA,
    ],
)
def _rbg(feat_hbm, spk_hbm, w1_hbm, b1_hbm, w2_hbm, b2_hbm, gate_hbm,
         spk_v, fb0, fb1, acc_v, gflat, w1v, b1v, w2v, b2v, semf0, semf1):
    wid = lax.axis_index("s") * NC + lax.axis_index("c")

    ih = [pltpu.async_copy(spk_hbm, spk_v, semf0),
          pltpu.async_copy(w1_hbm, w1v, semf0),
          pltpu.async_copy(b1_hbm, b1v, semf0),
          pltpu.async_copy(w2_hbm, w2v, semf0),
          pltpu.async_copy(b2_hbm, b2v, semf0)]
    for h in ih:
        h.wait()

    zero16 = jnp.zeros((16,), jnp.float32)
    fbufs = (fb0, fb1)
    fsems = (semf0, semf1)

    for w in range(ACCW // 16):
        acc_v[pl.ds(w * 16, 16)] = zero16

    def hsum(v):
        s = v[0]
        for l in range(1, 16):
            s = s + v[l]
        return s

    base = wid * ROWS_PER_W

    def mlp_gate(sums):
        means = [sums[k] * _INV_COUNTS[k] for k in range(NUM_BANDS)]
        h_chunks = []
        for c8 in range(HIDDEN // 16):
            hv = b1v[pl.ds(c8 * 16, 16)]
            for k in range(NUM_BANDS):
                hv = hv + means[k] * w1v[pl.ds(k * HIDDEN + c8 * 16, 16)]
            h_chunks.append(jnp.maximum(hv, 0.0))
        parts = [b2v[...], zero16, zero16, zero16]
        for c8 in range(HIDDEN // 16):
            for l in range(16):
                j = c8 * 16 + l
                parts[l % 4] = (
                    parts[l % 4] + h_chunks[c8][l] * w2v[pl.ds(j * 16, 16)])
        av = (parts[0] + parts[1]) + (parts[2] + parts[3])
        return 1.0 / (1.0 + jnp.exp(-av))

    def collect_sums():
        sums = []
        for k in range(NUM_BANDS):
            a = acc_v[pl.ds(k * 16, 16)]
            for b in range(1, NBANKS):
                a = a + acc_v[pl.ds(b * 96 + k * 16, 16)]
            sums.append(hsum(a))
        for w in range(ACCW // 16):
            acc_v[pl.ds(w * 16, 16)] = zero16
        return sums

    def row_body(r, carry):
        row = base + r
        for p in range(PIECES):
            pltpu.make_async_copy(
                feat_hbm.at[row, pl.ds(p * PW, PW)],
                fbufs[p % 2], fsems[p % 2]).wait()
            fb = fbufs[p % 2]

            def pair_body(i, _p=p, _fb=fb):
                off = i * 32
                pk = spk_v[pl.ds(_p * (PW // 2) + i * 16, 16)]
                s0 = pk & 0xFFFF
                s1 = lax.shift_right_logical(pk, 16)
                fv0 = _fb[pl.ds(off, 16)]
                fv1 = _fb[pl.ds(off + 16, 16)]
                b0 = (i & 3) * 192
                plsc.addupdate_scatter(acc_v, [s0 + b0], fv0)
                plsc.addupdate_scatter(acc_v, [s1 + (b0 + 96)], fv1)

            plsc.parallel_loop(0, PAIRS, unroll=8)(pair_body)

            if p + 2 < PIECES:
                pltpu.async_copy(
                    feat_hbm.at[row, pl.ds((p + 2) * PW, PW)],
                    fbufs[p % 2], fsems[p % 2])
            else:

                @pl.when(r < ROWS_PER_W - 1)
                def _issue_next(_p=p, _row=row):
                    q = _p + 2 - PIECES
                    pltpu.async_copy(
                        feat_hbm.at[_row + 1, pl.ds(q * PW, PW)],
                        fbufs[q % 2], fsems[q % 2])

        gflat[pl.ds(r * 16, 16)] = mlp_gate(collect_sums())
        return carry

    pltpu.async_copy(feat_hbm.at[base, pl.ds(0, PW)], fbufs[0], fsems[0])
    pltpu.async_copy(feat_hbm.at[base, pl.ds(PW, PW)], fbufs[1], fsems[1])
    lax.fori_loop(0, ROWS_PER_W, row_body, 0)
    pltpu.sync_copy(gflat, gate_hbm.at[pl.ds(wid * (ROWS_PER_W * 16),
                                             ROWS_PER_W * 16)])


def _expand_body(g_ref, m_ref, o_ref):
    o_ref[...] = jnp.dot(g_ref[...], m_ref[...],
                         preferred_element_type=jnp.float32)


_expand_tc = pl.pallas_call(
    _expand_body,
    grid=(pl.cdiv(F, TCB),),
    in_specs=[
        pl.BlockSpec((ROWS, 16), lambda i: (0, 0)),
        pl.BlockSpec((16, TCB), lambda i: (0, i)),
    ],
    out_specs=pl.BlockSpec((ROWS, TCB), lambda i: (0, i)),
    out_shape=jax.ShapeDtypeStruct((ROWS, F), jnp.float32),
)


def kernel(feat_flat, W1, b1, W2, b2):
    B, C, Fdim = feat_flat.shape
    feat2 = feat_flat.reshape(B * C, Fdim)
    w2p = jnp.zeros((HIDDEN, 16), W2.dtype).at[:, :NUM_BANDS].set(W2)
    b2p = jnp.zeros((16,), b2.dtype).at[:NUM_BANDS].set(b2)
    gates = _rbg(feat2, jnp.asarray(_SPK_NP), W1.reshape(-1), b1,
                 w2p.reshape(-1), b2p)
    out = _expand_tc(gates.reshape(ROWS, 16), jnp.asarray(_ONEHOT_NP))
    return out.reshape(B, C, Fdim)


# speedup vs baseline: 1.0081x; 1.0008x over previous
import functools

import numpy as np
import jax
import jax.numpy as jnp
from jax import lax
from jax.experimental import pallas as pl
from jax.experimental.pallas import tpu as pltpu
from jax.experimental.pallas import tpu_sc as plsc

H_FFT = 384
W_FFT = 193
NUM_BANDS = 6
HIDDEN = 128
F = H_FFT * W_FFT
ROWS = 4 * 96
NC, NS = 2, 16
NW = NC * NS
ROWS_PER_W = ROWS // NW
PIECES = 4
PW = F // PIECES
PAIRS = PW // 32
NBANKS = 8
ACCW = NBANKS * 96
TCB = 4096


def _band_tables():
    yy = np.arange(H_FFT, dtype=np.float32).reshape(-1, 1)
    xx = np.arange(W_FFT, dtype=np.float32).reshape(1, -1)
    ry = yy / max(H_FFT - 1, 1)
    rx = xx / max(W_FFT - 1, 1)
    r = np.sqrt(ry ** 2 + rx ** 2)
    r = r / (r.max() + 1e-8)
    band = np.minimum(np.floor(r * NUM_BANDS), NUM_BANDS - 1)
    band = band.astype(np.int32).reshape(-1)
    counts = np.zeros(NUM_BANDS, dtype=np.float32)
    for b in range(NUM_BANDS):
        counts[b] = max(float((band == b).sum()), 1.0)
    inv = np.float32(1.0) / (counts + np.float32(1e-6))
    sidx = band * 16 + (np.arange(F, dtype=np.int32) % 16)
    s = sidx.reshape(-1, 2, 16)
    spk = (s[:, 0, :] | (s[:, 1, :] << 16)).reshape(-1).astype(np.int32)
    onehot = np.zeros((16, F), dtype=np.float32)
    onehot[band, np.arange(F)] = 1.0
    return spk, [float(v) for v in inv], onehot


_SPK_NP, _INV_COUNTS, _ONEHOT_NP = _band_tables()

_MESH = plsc.VectorSubcoreMesh(core_axis_name="c", subcore_axis_name="s")


@functools.partial(
    pl.kernel,
    out_type=jax.ShapeDtypeStruct((ROWS * 16,), jnp.float32),
    mesh=_MESH,
    compiler_params=pltpu.CompilerParams(
        use_tc_tiling_on_sc=False, needs_layout_passes=False),
    scratch_types=[
        pltpu.VMEM((F // 2,), jnp.int32),
        pltpu.VMEM((PW,), jnp.float32),
        pltpu.VMEM((PW,), jnp.float32),
        pltpu.VMEM((ACCW,), jnp.float32),
        pltpu.VMEM((ROWS_PER_W * 16,), jnp.float32),
        pltpu.VMEM((NUM_BANDS * HIDDEN,), jnp.float32),
        pltpu.VMEM((HIDDEN,), jnp.float32),
        pltpu.VMEM((HIDDEN * 16,), jnp.float32),
        pltpu.VMEM((16,), jnp.float32),
        pltpu.SemaphoreType.DMA,
        pltpu.SemaphoreType.DMA,
    ],
)
def _rbg(feat_hbm, spk_hbm, w1_hbm, b1_hbm, w2_hbm, b2_hbm, gate_hbm,
         spk_v, fb0, fb1, acc_v, gflat, w1v, b1v, w2v, b2v, semf0, semf1):
    wid = lax.axis_index("s") * NC + lax.axis_index("c")

    ih = [pltpu.async_copy(spk_hbm, spk_v, semf0),
          pltpu.async_copy(w1_hbm, w1v, semf0),
          pltpu.async_copy(b1_hbm, b1v, semf0),
          pltpu.async_copy(w2_hbm, w2v, semf0),
          pltpu.async_copy(b2_hbm, b2v, semf0)]
    for h in ih:
        h.wait()

    zero16 = jnp.zeros((16,), jnp.float32)
    fbufs = (fb0, fb1)
    fsems = (semf0, semf1)

    for w in range(ACCW // 16):
        acc_v[pl.ds(w * 16, 16)] = zero16

    def hsum(v):
        s = v[0]
        for l in range(1, 16):
            s = s + v[l]
        return s

    base = wid * ROWS_PER_W

    def mlp_gate(sums):
        means = [sums[k] * _INV_COUNTS[k] for k in range(NUM_BANDS)]
        h_chunks = []
        for c8 in range(HIDDEN // 16):
            hv = b1v[pl.ds(c8 * 16, 16)]
            for k in range(NUM_BANDS):
                hv = hv + means[k] * w1v[pl.ds(k * HIDDEN + c8 * 16, 16)]
            h_chunks.append(jnp.maximum(hv, 0.0))
        parts = [b2v[...], zero16, zero16, zero16]
        for c8 in range(HIDDEN // 16):
            for l in range(16):
                j = c8 * 16 + l
                parts[l % 4] = (
                    parts[l % 4] + h_chunks[c8][l] * w2v[pl.ds(j * 16, 16)])
        av = (parts[0] + parts[1]) + (parts[2] + parts[3])
        return 1.0 / (1.0 + jnp.exp(-av))

    def collect_sums():
        sums = []
        for k in range(NUM_BANDS):
            a = acc_v[pl.ds(k * 16, 16)]
            for b in range(1, NBANKS):
                a = a + acc_v[pl.ds(b * 96 + k * 16, 16)]
            sums.append(hsum(a))
        for w in range(ACCW // 16):
            acc_v[pl.ds(w * 16, 16)] = zero16
        return sums

    def row_body(r, carry):
        row = base + r
        for p in range(PIECES):
            pltpu.make_async_copy(
                feat_hbm.at[row, pl.ds(p * PW, PW)],
                fbufs[p % 2], fsems[p % 2]).wait()
            fb = fbufs[p % 2]

            def pair_body(i, _p=p, _fb=fb):
                off = i * 32
                pk = spk_v[pl.ds(_p * (PW // 2) + i * 16, 16)]
                s0 = pk & 0xFFFF
                s1 = lax.shift_right_logical(pk, 16)
                fv0 = _fb[pl.ds(off, 16)]
                fv1 = _fb[pl.ds(off + 16, 16)]
                b0 = (i & 3) * 192
                plsc.addupdate_scatter(acc_v, [s0 + b0], fv0)
                plsc.addupdate_scatter(acc_v, [s1 + (b0 + 96)], fv1)

            plsc.parallel_loop(0, PAIRS, unroll=4)(pair_body)

            if p + 2 < PIECES:
                pltpu.async_copy(
                    feat_hbm.at[row, pl.ds((p + 2) * PW, PW)],
                    fbufs[p % 2], fsems[p % 2])
            else:

                @pl.when(r < ROWS_PER_W - 1)
                def _issue_next(_p=p, _row=row):
                    q = _p + 2 - PIECES
                    pltpu.async_copy(
                        feat_hbm.at[_row + 1, pl.ds(q * PW, PW)],
                        fbufs[q % 2], fsems[q % 2])

        gflat[pl.ds(r * 16, 16)] = mlp_gate(collect_sums())
        return carry

    pltpu.async_copy(feat_hbm.at[base, pl.ds(0, PW)], fbufs[0], fsems[0])
    pltpu.async_copy(feat_hbm.at[base, pl.ds(PW, PW)], fbufs[1], fsems[1])
    lax.fori_loop(0, ROWS_PER_W, row_body, 0)
    pltpu.sync_copy(gflat, gate_hbm.at[pl.ds(wid * (ROWS_PER_W * 16),
                                             ROWS_PER_W * 16)])


def _expand_body(g_ref, m_ref, o_ref):
    o_ref[...] = jnp.dot(g_ref[...], m_ref[...],
                         preferred_element_type=jnp.float32)


_expand_tc = pl.pallas_call(
    _expand_body,
    grid=(pl.cdiv(F, TCB),),
    in_specs=[
        pl.BlockSpec((ROWS, 16), lambda i: (0, 0)),
        pl.BlockSpec((16, TCB), lambda i: (0, i)),
    ],
    out_specs=pl.BlockSpec((ROWS, TCB), lambda i: (0, i)),
    out_shape=jax.ShapeDtypeStruct((ROWS, F), jnp.float32),
)


def kernel(feat_flat, W1, b1, W2, b2):
    B, C, Fdim = feat_flat.shape
    feat2 = feat_flat.reshape(B * C, Fdim)
    w2p = jnp.zeros((HIDDEN, 16), W2.dtype).at[:, :NUM_BANDS].set(W2)
    b2p = jnp.zeros((16,), b2.dtype).at[:NUM_BANDS].set(b2)
    gates = _rbg(feat2, jnp.asarray(_SPK_NP), W1.reshape(-1), b1,
                 w2p.reshape(-1), b2p)
    out = _expand_tc(gates.reshape(ROWS, 16), jnp.asarray(_ONEHOT_NP))
    return out.reshape(B, C, Fdim)
